# baseline (speedup 1.0000x reference)
import jax
import jax.numpy as jnp
from jax import lax
from jax.experimental import pallas as pl
from jax.experimental.pallas import tpu as pltpu

_B = 32
_P = 16800
_NO = 32
_TH = 0.35
_V0 = 0.1
_V1 = 0.2
_NEGPOS = 7
_ROWS = 8


def _smooth_l1(d):
    a = jnp.abs(d)
    return jnp.where(a < 1.0, 0.5 * d * d, a - 0.5)


def _match_body(pr_ref, tgt_ref, s_out):
    pr = pr_ref[:, :]
    cx, cy, w, h = pr[0:1], pr[1:2], pr[2:3], pr[3:4]
    px1 = cx - w * 0.5
    py1 = cy - h * 0.5
    px2 = cx + w * 0.5
    py2 = cy + h * 0.5
    area_p = w * h
    iota_j = lax.broadcasted_iota(jnp.int32, (_NO, _P), 0)
    iota_p = lax.broadcasted_iota(jnp.int32, (_NO, _P), 1)

    for i in range(_ROWS):
        tg = tgt_ref[i]
        tx1, ty1, tx2, ty2 = tg[:, 0:1], tg[:, 1:2], tg[:, 2:3], tg[:, 3:4]

        ix = jnp.clip(jnp.minimum(tx2, px2) - jnp.maximum(tx1, px1), 0.0, None)
        iy = jnp.clip(jnp.minimum(ty2, py2) - jnp.maximum(ty1, py1), 0.0, None)
        inter = ix * iy
        area_t = (tx2 - tx1) * (ty2 - ty1)
        ov = inter / (area_t + area_p - inter)

        bto = jnp.max(ov, axis=0, keepdims=True)
        bpo = jnp.max(ov, axis=1, keepdims=True)
        bpi = jnp.min(jnp.where(ov == bpo, iota_p, _P),
                      axis=1, keepdims=True)
        valid = (bpo >= 0.2).astype(jnp.int32)

        eq = bpi == iota_p
        ismax = ov == bto
        a_j = (iota_j + 64) * 2 + valid
        b_j = (63 - iota_j) * 2
        score = jnp.where(eq, a_j, jnp.where(ismax, b_j, 0))
        sc = jnp.max(score, axis=0, keepdims=True)
        s_out[0, pl.ds(i, 1), :] = sc * 2 + (bto >= _TH).astype(jnp.int32)


def _loss_body(s_ref, loc_ref, conf_ref, lm_ref, pr_ref, tgt_t_ref,
               out_ref, lc_scr, np_scr, acc_ref):
    b = pl.program_id(0)

    pr = pr_ref[:, :]
    cx, cy, w, h = pr[0:1], pr[1:2], pr[2:3], pr[3:4]
    rcpw = 1.0 / (_V0 * w)
    rcph = 1.0 / (_V0 * h)
    winv = rcpw * _V0
    hinv = rcph * _V0
    pcs = jnp.concatenate([cx, cy] * 5, axis=0)
    psr = jnp.concatenate([rcpw, rcph] * 5, axis=0)
    iota_j = lax.broadcasted_iota(jnp.int32, (_NO, _P), 0)

    def one_row(i):
        sp = s_ref[0, pl.ds(i, 1), :]
        bth = (sp & 1) == 1
        s = sp >> 1
        forced = s >= 128
        sh = s // 2
        bti2 = jnp.where(forced, sh - 64, 63 - sh)
        fpos = forced & ((s & 1) == 1)

        oh = (iota_j == bti2).astype(jnp.float32)
        g = jnp.dot(tgt_t_ref[i], oh, preferred_element_type=jnp.float32)

        lab = g[14:15]
        pos = (fpos | bth) & (lab != 0.0)

        g0, g1, g2, g3 = g[0:1], g[1:2], g[2:3], g[3:4]
        ecx = ((g0 + g2) * 0.5 - cx) * rcpw
        ecy = ((g1 + g3) * 0.5 - cy) * rcph
        ew = jnp.log((g2 - g0) * winv) * (1.0 / _V1)
        eh = jnp.log((g3 - g1) * hinv) * (1.0 / _V1)
        enc = jnp.concatenate([ecx, ecy, ew, eh], axis=0)
        loc_sum = jnp.sum(jnp.where(pos, _smooth_l1(loc_ref[i] - enc), 0.0))

        elm = (g[4:14] - pcs) * psr
        lmask = pos & ~jnp.isnan(elm)
        lm_sum = jnp.sum(jnp.where(lmask, _smooth_l1(lm_ref[i] - elm), 0.0))

        cf = conf_ref[i]
        c0, c1 = cf[0:1], cf[1:2]
        mx = jnp.maximum(c0, c1)
        lse2 = mx + jnp.log(1.0 + jnp.exp(-jnp.abs(c0 - c1)))
        cepos = jnp.sum(jnp.where(pos, lse2 - c1, 0.0))
        lc = jnp.where(pos, 0.0, lse2 - c0)
        npos = jnp.sum(pos.astype(jnp.float32))
        return loc_sum, lm_sum, cepos, npos, lc

    res = [one_row(i) for i in range(_ROWS)]
    for i, (_, _, _, npos_i, lc_i) in enumerate(res):
        lc_scr[pl.ds(b * _ROWS + i, 1), :] = lc_i
        np_scr[pl.ds(b * _ROWS + i, 1), :] = jnp.full((1, 128), npos_i,
                                                      jnp.float32)

    @pl.when(b == 0)
    def _init():
        acc_ref[0] = 0.0
        acc_ref[1] = 0.0
        acc_ref[2] = 0.0

    acc_ref[0] = acc_ref[0] + sum(r[0] for r in res)
    acc_ref[1] = acc_ref[1] + sum(r[1] for r in res)
    acc_ref[2] = acc_ref[2] + sum(r[2] for r in res)

    @pl.when(b == _B // _ROWS - 1)
    def _finish():
        lcv = lc_scr[:, :]
        lci = lax.bitcast_convert_type(lcv, jnp.int32)
        npv = np_scr[:, 0:1]
        k = jnp.minimum(npv.astype(jnp.int32) * _NEGPOS, _P - 1)

        bits = jnp.zeros((_B, 1), jnp.int32)
        for bit in range(30, -1, -1):
            cand = bits | (1 << bit)
            cnt = jnp.sum((lci >= cand).astype(jnp.int32),
                          axis=1, keepdims=True)
            bits = jnp.where(cnt >= k, cand, bits)

        tf = lax.bitcast_convert_type(bits, jnp.float32)
        gt = lci > bits
        mgt = jnp.sum(jnp.where(gt, 1.0, 0.0), axis=1, keepdims=True)
        sgt = jnp.sum(jnp.where(gt, lcv, 0.0), axis=1, keepdims=True)
        nsum = jnp.where(k > 0, sgt + (k.astype(jnp.float32) - mgt) * tf, 0.0)

        ce_total = acc_ref[2] + jnp.sum(nsum)
        ntot = jnp.sum(npv)
        rows = jnp.concatenate(
            [jnp.full((1, 128), acc_ref[0], jnp.float32),
             jnp.full((1, 128), ce_total, jnp.float32),
             jnp.full((1, 128), acc_ref[1], jnp.float32),
             jnp.full((1, 128), ntot, jnp.float32),
             jnp.zeros((4, 128), jnp.float32)], axis=0)
        out_ref[:, :] = rows


def kernel(locations_data, confidence_data, landmark_data, priors, targets):
    loc_t = jnp.transpose(locations_data, (0, 2, 1))
    conf_t = jnp.transpose(confidence_data, (0, 2, 1))
    lm_t = jnp.transpose(landmark_data, (0, 2, 1))
    pr_t = jnp.transpose(priors, (1, 0))
    tgt_t = jnp.transpose(targets, (0, 2, 1))

    s_all = pl.pallas_call(
        _match_body,
        grid=(_B // _ROWS,),
        in_specs=[
            pl.BlockSpec((4, _P), lambda b: (0, 0)),
            pl.BlockSpec((_ROWS, _NO, 15), lambda b: (b, 0, 0)),
        ],
        out_specs=pl.BlockSpec((1, _ROWS, _P), lambda b: (b, 0, 0)),
        out_shape=jax.ShapeDtypeStruct((_B // _ROWS, _ROWS, _P), jnp.int32),
        compiler_params=pltpu.CompilerParams(
            dimension_semantics=("arbitrary",)),
    )(pr_t, targets)

    out = pl.pallas_call(
        _loss_body,
        grid=(_B // _ROWS,),
        in_specs=[
            pl.BlockSpec((1, _ROWS, _P), lambda b: (b, 0, 0)),
            pl.BlockSpec((_ROWS, 4, _P), lambda b: (b, 0, 0)),
            pl.BlockSpec((_ROWS, 2, _P), lambda b: (b, 0, 0)),
            pl.BlockSpec((_ROWS, 10, _P), lambda b: (b, 0, 0)),
            pl.BlockSpec((4, _P), lambda b: (0, 0)),
            pl.BlockSpec((_ROWS, 15, _NO), lambda b: (b, 0, 0)),
        ],
        out_specs=pl.BlockSpec((8, 128), lambda b: (0, 0)),
        out_shape=jax.ShapeDtypeStruct((8, 128), jnp.float32),
        scratch_shapes=[
            pltpu.VMEM((_B, _P), jnp.float32),
            pltpu.VMEM((_B, 128), jnp.float32),
            pltpu.SMEM((4,), jnp.float32),
        ],
        compiler_params=pltpu.CompilerParams(
            dimension_semantics=("arbitrary",)),
    )(s_all, loc_t, conf_t, lm_t, pr_t, tgt_t)

    n = jnp.maximum(out[3, 0], 1.0)
    return (out[0, 0] / n, out[1, 0] / n, out[2, 0] / n)


# speedup vs baseline: 1.0684x; 1.0684x over previous
import jax
import jax.numpy as jnp
from jax import lax
from jax.experimental import pallas as pl
from jax.experimental.pallas import tpu as pltpu

_B = 32
_P = 16800
_NO = 32
_TH = 0.35
_V0 = 0.1
_V1 = 0.2
_NEGPOS = 7
_ROWS = 4


def _smooth_l1(d):
    a = jnp.abs(d)
    return jnp.where(a < 1.0, 0.5 * d * d, a - 0.5)


def _match_body(pr_ref, tgt_ref, s_out):
    pr = pr_ref[:, :]
    cx, cy, w, h = pr[0:1], pr[1:2], pr[2:3], pr[3:4]
    px1 = cx - w * 0.5
    py1 = cy - h * 0.5
    px2 = cx + w * 0.5
    py2 = cy + h * 0.5
    area_p = w * h
    iota_j = lax.broadcasted_iota(jnp.int32, (_NO, _P), 0)
    iota_p = lax.broadcasted_iota(jnp.int32, (_NO, _P), 1)

    for i in range(_ROWS):
        tg = tgt_ref[i]
        tx1, ty1, tx2, ty2 = tg[:, 0:1], tg[:, 1:2], tg[:, 2:3], tg[:, 3:4]

        ix = jnp.clip(jnp.minimum(tx2, px2) - jnp.maximum(tx1, px1), 0.0, None)
        iy = jnp.clip(jnp.minimum(ty2, py2) - jnp.maximum(ty1, py1), 0.0, None)
        inter = ix * iy
        area_t = (tx2 - tx1) * (ty2 - ty1)
        ov = inter / (area_t + area_p - inter)

        bto = jnp.max(ov, axis=0, keepdims=True)
        bpo = jnp.max(ov, axis=1, keepdims=True)
        bpi = jnp.min(jnp.where(ov == bpo, iota_p, _P),
                      axis=1, keepdims=True)
        valid = (bpo >= 0.2).astype(jnp.int32)

        eq = bpi == iota_p
        ismax = ov == bto
        a_j = (iota_j + 64) * 2 + valid
        b_j = (63 - iota_j) * 2
        score = jnp.where(eq, a_j, jnp.where(ismax, b_j, 0))
        sc = jnp.max(score, axis=0, keepdims=True)
        s_out[0, pl.ds(i, 1), :] = sc * 2 + (bto >= _TH).astype(jnp.int32)


def _loss_body(s_ref, loc_ref, conf_ref, lm_ref, pr_ref, tgt_t_ref,
               out_ref, lc_scr, np_scr, acc_ref):
    b = pl.program_id(0)

    pr = pr_ref[:, :]
    cx, cy, w, h = pr[0:1], pr[1:2], pr[2:3], pr[3:4]
    rcpw = 1.0 / (_V0 * w)
    rcph = 1.0 / (_V0 * h)
    winv = rcpw * _V0
    hinv = rcph * _V0
    pcs = jnp.concatenate([cx, cy] * 5, axis=0)
    psr = jnp.concatenate([rcpw, rcph] * 5, axis=0)
    iota_j = lax.broadcasted_iota(jnp.int32, (_NO, _P), 0)

    def one_row(i):
        sp = s_ref[0, pl.ds(i, 1), :]
        bth = (sp & 1) == 1
        s = sp >> 1
        forced = s >= 128
        sh = s // 2
        bti2 = jnp.where(forced, sh - 64, 63 - sh)
        fpos = forced & ((s & 1) == 1)

        oh = (iota_j == bti2).astype(jnp.float32)
        g = jnp.dot(tgt_t_ref[i], oh, preferred_element_type=jnp.float32)

        lab = g[14:15]
        pos = (fpos | bth) & (lab != 0.0)

        g0, g1, g2, g3 = g[0:1], g[1:2], g[2:3], g[3:4]
        ecx = ((g0 + g2) * 0.5 - cx) * rcpw
        ecy = ((g1 + g3) * 0.5 - cy) * rcph
        ew = jnp.log((g2 - g0) * winv) * (1.0 / _V1)
        eh = jnp.log((g3 - g1) * hinv) * (1.0 / _V1)
        enc = jnp.concatenate([ecx, ecy, ew, eh], axis=0)
        loc_sum = jnp.sum(jnp.where(pos, _smooth_l1(loc_ref[i] - enc), 0.0))

        elm = (g[4:14] - pcs) * psr
        lmask = pos & ~jnp.isnan(elm)
        lm_sum = jnp.sum(jnp.where(lmask, _smooth_l1(lm_ref[i] - elm), 0.0))

        cf = conf_ref[i]
        c0, c1 = cf[0:1], cf[1:2]
        mx = jnp.maximum(c0, c1)
        lse2 = mx + jnp.log(jnp.exp(c0 - mx) + jnp.exp(c1 - mx))
        cepos = jnp.sum(jnp.where(pos, lse2 - c1, 0.0))
        lc = jnp.where(pos, 0.0, lse2 - c0)
        npos = jnp.sum(pos.astype(jnp.float32))
        return loc_sum, lm_sum, cepos, npos, lc

    res = [one_row(i) for i in range(_ROWS)]
    for i, (_, _, _, npos_i, lc_i) in enumerate(res):
        lc_scr[pl.ds(b * _ROWS + i, 1), :] = lc_i
        np_scr[pl.ds(b * _ROWS + i, 1), :] = jnp.full((1, 128), npos_i,
                                                      jnp.float32)

    @pl.when(b == 0)
    def _init():
        acc_ref[0] = 0.0
        acc_ref[1] = 0.0
        acc_ref[2] = 0.0

    acc_ref[0] = acc_ref[0] + sum(r[0] for r in res)
    acc_ref[1] = acc_ref[1] + sum(r[1] for r in res)
    acc_ref[2] = acc_ref[2] + sum(r[2] for r in res)

    @pl.when(b == _B // _ROWS - 1)
    def _finish():
        lcv = lc_scr[:, :]
        lci = lax.bitcast_convert_type(lcv, jnp.int32)
        npv = np_scr[:, 0:1]
        k = jnp.minimum(npv.astype(jnp.int32) * _NEGPOS, _P - 1)

        bits = jnp.zeros((_B, 1), jnp.int32)
        for bit in range(30, -1, -1):
            cand = bits | (1 << bit)
            cnt = jnp.sum((lci >= cand).astype(jnp.int32),
                          axis=1, keepdims=True)
            bits = jnp.where(cnt >= k, cand, bits)

        tf = lax.bitcast_convert_type(bits, jnp.float32)
        gt = lci > bits
        mgt = jnp.sum(jnp.where(gt, 1.0, 0.0), axis=1, keepdims=True)
        sgt = jnp.sum(jnp.where(gt, lcv, 0.0), axis=1, keepdims=True)
        nsum = jnp.where(k > 0, sgt + (k.astype(jnp.float32) - mgt) * tf, 0.0)

        ce_total = acc_ref[2] + jnp.sum(nsum)
        ntot = jnp.sum(npv)
        rows = jnp.concatenate(
            [jnp.full((1, 128), acc_ref[0], jnp.float32),
             jnp.full((1, 128), ce_total, jnp.float32),
             jnp.full((1, 128), acc_ref[1], jnp.float32),
             jnp.full((1, 128), ntot, jnp.float32),
             jnp.zeros((4, 128), jnp.float32)], axis=0)
        out_ref[:, :] = rows


def kernel(locations_data, confidence_data, landmark_data, priors, targets):
    loc_t = jnp.transpose(locations_data, (0, 2, 1))
    conf_t = jnp.transpose(confidence_data, (0, 2, 1))
    lm_t = jnp.transpose(landmark_data, (0, 2, 1))
    pr_t = jnp.transpose(priors, (1, 0))
    tgt_t = jnp.transpose(targets, (0, 2, 1))

    s_all = pl.pallas_call(
        _match_body,
        grid=(_B // _ROWS,),
        in_specs=[
            pl.BlockSpec((4, _P), lambda b: (0, 0)),
            pl.BlockSpec((_ROWS, _NO, 15), lambda b: (b, 0, 0)),
        ],
        out_specs=pl.BlockSpec((1, _ROWS, _P), lambda b: (b, 0, 0)),
        out_shape=jax.ShapeDtypeStruct((_B // _ROWS, _ROWS, _P), jnp.int32),
        compiler_params=pltpu.CompilerParams(
            dimension_semantics=("arbitrary",)),
    )(pr_t, targets)

    out = pl.pallas_call(
        _loss_body,
        grid=(_B // _ROWS,),
        in_specs=[
            pl.BlockSpec((1, _ROWS, _P), lambda b: (b, 0, 0)),
            pl.BlockSpec((_ROWS, 4, _P), lambda b: (b, 0, 0)),
            pl.BlockSpec((_ROWS, 2, _P), lambda b: (b, 0, 0)),
            pl.BlockSpec((_ROWS, 10, _P), lambda b: (b, 0, 0)),
            pl.BlockSpec((4, _P), lambda b: (0, 0)),
            pl.BlockSpec((_ROWS, 15, _NO), lambda b: (b, 0, 0)),
        ],
        out_specs=pl.BlockSpec((8, 128), lambda b: (0, 0)),
        out_shape=jax.ShapeDtypeStruct((8, 128), jnp.float32),
        scratch_shapes=[
            pltpu.VMEM((_B, _P), jnp.float32),
            pltpu.VMEM((_B, 128), jnp.float32),
            pltpu.SMEM((4,), jnp.float32),
        ],
        compiler_params=pltpu.CompilerParams(
            dimension_semantics=("arbitrary",)),
    )(s_all, loc_t, conf_t, lm_t, pr_t, tgt_t)

    n = jnp.maximum(out[3, 0], 1.0)
    return (out[0, 0] / n, out[1, 0] / n, out[2, 0] / n)


# speedup vs baseline: 1.0783x; 1.0093x over previous
import jax
import jax.numpy as jnp
from jax import lax
from jax.experimental import pallas as pl
from jax.experimental.pallas import tpu as pltpu

_B = 32
_P = 16800
_NO = 32
_TH = 0.35
_V0 = 0.1
_V1 = 0.2
_NEGPOS = 7
_ROWS = 4


def _smooth_l1(d):
    a = jnp.abs(d)
    return jnp.where(a < 1.0, 0.5 * d * d, a - 0.5)


def _match_body(pr_ref, tgt_ref, s_out, bto_out):
    pr = pr_ref[:, :]
    cx, cy, w, h = pr[0:1], pr[1:2], pr[2:3], pr[3:4]
    px1 = cx - w * 0.5
    py1 = cy - h * 0.5
    px2 = cx + w * 0.5
    py2 = cy + h * 0.5
    area_p = w * h
    iota_j = lax.broadcasted_iota(jnp.int32, (_NO, _P), 0)
    iota_p = lax.broadcasted_iota(jnp.int32, (_NO, _P), 1)

    for i in range(_ROWS):
        tg = tgt_ref[i]
        tx1, ty1, tx2, ty2 = tg[:, 0:1], tg[:, 1:2], tg[:, 2:3], tg[:, 3:4]

        ix = jnp.clip(jnp.minimum(tx2, px2) - jnp.maximum(tx1, px1), 0.0, None)
        iy = jnp.clip(jnp.minimum(ty2, py2) - jnp.maximum(ty1, py1), 0.0, None)
        inter = ix * iy
        area_t = (tx2 - tx1) * (ty2 - ty1)
        ov = inter / (area_t + area_p - inter)

        bto = jnp.max(ov, axis=0, keepdims=True)
        bpo = jnp.max(ov, axis=1, keepdims=True)
        bpi = jnp.min(jnp.where(ov == bpo, iota_p, _P),
                      axis=1, keepdims=True)
        valid = (bpo >= 0.2).astype(jnp.int32)

        eq = bpi == iota_p
        ismax = ov == bto
        a_j = (iota_j + 64) * 2 + valid
        b_j = (63 - iota_j) * 2
        score = jnp.where(eq, a_j, jnp.where(ismax, b_j, 0))
        s_out[0, pl.ds(i, 1), :] = jnp.max(score, axis=0, keepdims=True)
        bto_out[0, pl.ds(i, 1), :] = bto


def _loss_body(s_ref, bto_ref, loc_ref, conf_ref, lm_ref, pr_ref, tgt_t_ref,
               out_ref, lc_scr, np_scr, acc_ref):
    b = pl.program_id(0)

    pr = pr_ref[:, :]
    cx, cy, w, h = pr[0:1], pr[1:2], pr[2:3], pr[3:4]
    iota_j = lax.broadcasted_iota(jnp.int32, (_NO, _P), 0)

    def one_row(i):
        s = s_ref[0, pl.ds(i, 1), :]
        bto = bto_ref[0, pl.ds(i, 1), :]
        forced = s >= 128
        sh = s // 2
        bti2 = jnp.where(forced, sh - 64, 63 - sh)
        fpos = forced & ((s & 1) == 1)

        oh = (iota_j == bti2).astype(jnp.float32)
        g = jnp.dot(tgt_t_ref[i], oh, preferred_element_type=jnp.float32)

        lab = g[14:15]
        pos = (fpos | (bto >= _TH)) & (lab != 0.0)

        g0, g1, g2, g3 = g[0:1], g[1:2], g[2:3], g[3:4]
        ecx = ((g0 + g2) * 0.5 - cx) / (_V0 * w)
        ecy = ((g1 + g3) * 0.5 - cy) / (_V0 * h)
        ew = jnp.log((g2 - g0) / w) * (1.0 / _V1)
        eh = jnp.log((g3 - g1) / h) * (1.0 / _V1)
        enc = jnp.concatenate([ecx, ecy, ew, eh], axis=0)
        loc_sum = jnp.sum(jnp.where(pos, _smooth_l1(loc_ref[i] - enc), 0.0))

        pcs = jnp.concatenate([cx, cy] * 5, axis=0)
        pss = jnp.concatenate([w, h] * 5, axis=0)
        elm = (g[4:14] - pcs) / (_V0 * pss)
        lmask = pos & ~jnp.isnan(elm)
        lm_sum = jnp.sum(jnp.where(lmask, _smooth_l1(lm_ref[i] - elm), 0.0))

        cf = conf_ref[i]
        c0, c1 = cf[0:1], cf[1:2]
        mx = jnp.maximum(c0, c1)
        lse2 = mx + jnp.log(jnp.exp(c0 - mx) + jnp.exp(c1 - mx))
        cepos = jnp.sum(jnp.where(pos, lse2 - c1, 0.0))
        lc = jnp.where(pos, 0.0, lse2 - c0)
        npos = jnp.sum(pos.astype(jnp.float32))
        return loc_sum, lm_sum, cepos, npos, lc

    res = [one_row(i) for i in range(_ROWS)]
    for i, (_, _, _, npos_i, lc_i) in enumerate(res):
        lc_scr[pl.ds(b * _ROWS + i, 1), :] = lc_i
        np_scr[pl.ds(b * _ROWS + i, 1), :] = jnp.full((1, 128), npos_i,
                                                      jnp.float32)

    @pl.when(b == 0)
    def _init():
        acc_ref[0] = 0.0
        acc_ref[1] = 0.0
        acc_ref[2] = 0.0

    acc_ref[0] = acc_ref[0] + sum(r[0] for r in res)
    acc_ref[1] = acc_ref[1] + sum(r[1] for r in res)
    acc_ref[2] = acc_ref[2] + sum(r[2] for r in res)

    @pl.when(b == _B // _ROWS - 1)
    def _finish():
        lcv = lc_scr[:, :]
        lci = lax.bitcast_convert_type(lcv, jnp.int32)
        npv = np_scr[:, 0:1]
        k = jnp.minimum(npv.astype(jnp.int32) * _NEGPOS, _P - 1)

        bits = jnp.zeros((_B, 1), jnp.int32)
        for bit in range(30, -1, -1):
            cand = bits | (1 << bit)
            cnt = jnp.sum((lci >= cand).astype(jnp.int32),
                          axis=1, keepdims=True)
            bits = jnp.where(cnt >= k, cand, bits)

        tf = lax.bitcast_convert_type(bits, jnp.float32)
        gt = lci > bits
        mgt = jnp.sum(jnp.where(gt, 1.0, 0.0), axis=1, keepdims=True)
        sgt = jnp.sum(jnp.where(gt, lcv, 0.0), axis=1, keepdims=True)
        nsum = jnp.where(k > 0, sgt + (k.astype(jnp.float32) - mgt) * tf, 0.0)

        ce_total = acc_ref[2] + jnp.sum(nsum)
        ntot = jnp.sum(npv)
        rows = jnp.concatenate(
            [jnp.full((1, 128), acc_ref[0], jnp.float32),
             jnp.full((1, 128), ce_total, jnp.float32),
             jnp.full((1, 128), acc_ref[1], jnp.float32),
             jnp.full((1, 128), ntot, jnp.float32),
             jnp.zeros((4, 128), jnp.float32)], axis=0)
        out_ref[:, :] = rows


def kernel(locations_data, confidence_data, landmark_data, priors, targets):
    loc_t = jnp.transpose(locations_data, (0, 2, 1))
    conf_t = jnp.transpose(confidence_data, (0, 2, 1))
    lm_t = jnp.transpose(landmark_data, (0, 2, 1))
    pr_t = jnp.transpose(priors, (1, 0))
    tgt_t = jnp.transpose(targets, (0, 2, 1))

    s_all, bto_all = pl.pallas_call(
        _match_body,
        grid=(_B // _ROWS,),
        in_specs=[
            pl.BlockSpec((4, _P), lambda b: (0, 0)),
            pl.BlockSpec((_ROWS, _NO, 15), lambda b: (b, 0, 0)),
        ],
        out_specs=[
            pl.BlockSpec((1, _ROWS, _P), lambda b: (b, 0, 0)),
            pl.BlockSpec((1, _ROWS, _P), lambda b: (b, 0, 0)),
        ],
        out_shape=[
            jax.ShapeDtypeStruct((_B // _ROWS, _ROWS, _P), jnp.int32),
            jax.ShapeDtypeStruct((_B // _ROWS, _ROWS, _P), jnp.float32),
        ],
        compiler_params=pltpu.CompilerParams(
            dimension_semantics=("arbitrary",)),
    )(pr_t, targets)

    out = pl.pallas_call(
        _loss_body,
        grid=(_B // _ROWS,),
        in_specs=[
            pl.BlockSpec((1, _ROWS, _P), lambda b: (b, 0, 0)),
            pl.BlockSpec((1, _ROWS, _P), lambda b: (b, 0, 0)),
            pl.BlockSpec((_ROWS, 4, _P), lambda b: (b, 0, 0)),
            pl.BlockSpec((_ROWS, 2, _P), lambda b: (b, 0, 0)),
            pl.BlockSpec((_ROWS, 10, _P), lambda b: (b, 0, 0)),
            pl.BlockSpec((4, _P), lambda b: (0, 0)),
            pl.BlockSpec((_ROWS, 15, _NO), lambda b: (b, 0, 0)),
        ],
        out_specs=pl.BlockSpec((8, 128), lambda b: (0, 0)),
        out_shape=jax.ShapeDtypeStruct((8, 128), jnp.float32),
        scratch_shapes=[
            pltpu.VMEM((_B, _P), jnp.float32),
            pltpu.VMEM((_B, 128), jnp.float32),
            pltpu.SMEM((4,), jnp.float32),
        ],
        compiler_params=pltpu.CompilerParams(
            dimension_semantics=("arbitrary",)),
    )(s_all, bto_all, loc_t, conf_t, lm_t, pr_t, tgt_t)

    n = jnp.maximum(out[3, 0], 1.0)
    return (out[0, 0] / n, out[1, 0] / n, out[2, 0] / n)
